# baseline (speedup 1.0000x reference)
import functools

import jax
import jax.numpy as jnp
from jax import lax
from jax.experimental import pallas as pl
from jax.experimental.pallas import tpu as pltpu
from jax.experimental.pallas import tpu_sc as plsc

BATCH = 16384
NUM_FEATURES = 10
NUM_VALUES = 500
HIDDEN = 128

NUM_CORES = 2
NUM_SUBCORES = 16
NUM_WORKERS = NUM_CORES * NUM_SUBCORES
B_PER_W = BATCH // NUM_WORKERS
CHUNK = 128
N_CHUNKS = B_PER_W // CHUNK
LANES = 16
VECS_PER_ROW = HIDDEN // LANES
GROUPS_PER_CHUNK = CHUNK // LANES

TAB_ROWS = NUM_FEATURES * NUM_VALUES
STAGE_ROWS = TAB_ROWS // NUM_SUBCORES
STAGE_TAIL = TAB_ROWS - (NUM_SUBCORES - 1) * STAGE_ROWS


def _sc_encode(xi, tab):
    mesh = plsc.VectorSubcoreMesh(core_axis_name="c", subcore_axis_name="s")

    @functools.partial(
        pl.kernel,
        mesh=mesh,
        out_type=jax.ShapeDtypeStruct((BATCH, HIDDEN), jnp.float32),
        scratch_types=[
            pltpu.VMEM((NUM_FEATURES, N_CHUNKS, CHUNK), jnp.int32),
            pltpu.VMEM((4, CHUNK, HIDDEN), jnp.float32),
            pltpu.VMEM_SHARED((TAB_ROWS, HIDDEN), jnp.float32),
            pltpu.SemaphoreType.DMA,
            pltpu.SemaphoreType.DMA,
            pltpu.SemaphoreType.DMA((4,)),
            pltpu.SemaphoreType.DMA((4,)),
        ],
    )
    def k(xi_hbm, tab_hbm, out_hbm, idx_all, acc2, shared_tab,
          ssem, hsem, gsem, osem):
        wid = lax.axis_index("s") * NUM_CORES + lax.axis_index("c")
        sid = lax.axis_index("s")
        base = wid * B_PER_W

        @pl.when(sid < NUM_SUBCORES - 1)
        def _():
            pltpu.async_copy(
                tab_hbm.at[pl.ds(sid * STAGE_ROWS, STAGE_ROWS)],
                shared_tab.at[pl.ds(sid * STAGE_ROWS, STAGE_ROWS)],
                ssem,
            )

        @pl.when(sid == NUM_SUBCORES - 1)
        def _():
            pltpu.async_copy(
                tab_hbm.at[pl.ds((NUM_SUBCORES - 1) * STAGE_ROWS, STAGE_TAIL)],
                shared_tab.at[
                    pl.ds((NUM_SUBCORES - 1) * STAGE_ROWS, STAGE_TAIL)],
                ssem,
            )

        pltpu.sync_copy(xi_hbm.at[wid], idx_all)

        zero16 = jnp.zeros((LANES,), jnp.float32)

        def zero_acc(b):
            def zrow(i, _):
                for j in range(VECS_PER_ROW):
                    acc2.at[b][i, pl.ds(j * LANES, LANES)] = zero16
                return 0

            lax.fori_loop(0, CHUNK, zrow, 0)

        def fire_hbm_gathers(cc, b):
            def feat(f, _):
                pltpu.async_copy(
                    tab_hbm.at[idx_all.at[f, cc]], acc2.at[b], hsem,
                    add=True,
                )
                return 0

            lax.fori_loop(0, NUM_FEATURES, feat, 0)

        def fire_gathers(cc, b):
            def feat(f, _):
                pltpu.async_copy(
                    shared_tab.at[idx_all.at[f, cc]], acc2.at[b], gsem.at[b],
                    add=True,
                )
                return 0

            lax.fori_loop(0, NUM_FEATURES, feat, 0)

        def drain_gathers(cc, b):
            def feat_h(f, _):
                pltpu.make_async_copy(
                    tab_hbm.at[idx_all.at[f, cc]], acc2.at[b], hsem
                ).wait()
                return 0

            def feat_s(f, _):
                pltpu.make_async_copy(
                    shared_tab.at[idx_all.at[f, cc]], acc2.at[b], gsem.at[b]
                ).wait()
                return 0

            @pl.when(cc == 0)
            def _():
                lax.fori_loop(0, NUM_FEATURES, feat_h, 0)

            @pl.when(cc > 0)
            def _():
                lax.fori_loop(0, NUM_FEATURES, feat_s, 0)

        zero_acc(0)
        fire_hbm_gathers(0, 0)

        @pl.when(sid < NUM_SUBCORES - 1)
        def _():
            pltpu.make_async_copy(
                tab_hbm.at[pl.ds(sid * STAGE_ROWS, STAGE_ROWS)],
                shared_tab.at[pl.ds(sid * STAGE_ROWS, STAGE_ROWS)],
                ssem,
            ).wait()

        @pl.when(sid == NUM_SUBCORES - 1)
        def _():
            pltpu.make_async_copy(
                tab_hbm.at[pl.ds((NUM_SUBCORES - 1) * STAGE_ROWS, STAGE_TAIL)],
                shared_tab.at[
                    pl.ds((NUM_SUBCORES - 1) * STAGE_ROWS, STAGE_TAIL)],
                ssem,
            ).wait()

        plsc.subcore_barrier()

        def prime(c, _):
            zero_acc(c)
            fire_gathers(c, c)
            return 0

        lax.fori_loop(1, N_CHUNKS, prime, 0)

        def chunk_body(c, _):
            drain_gathers(c, c)
            pltpu.async_copy(
                acc2.at[c], out_hbm.at[pl.ds(base + c * CHUNK, CHUNK)],
                osem.at[c],
            )
            return 0

        lax.fori_loop(0, N_CHUNKS, chunk_body, 0)

        for c in range(N_CHUNKS):
            pltpu.make_async_copy(
                acc2.at[c],
                out_hbm.at[pl.ds(base + c * CHUNK, CHUNK)],
                osem.at[c],
            ).wait()

    return k(xi, tab)


def kernel(x, tables):
    if x.ndim == 1:
        x = x[:, None]
    xi = x.astype(jnp.int32) + NUM_VALUES * jnp.arange(
        NUM_FEATURES, dtype=jnp.int32)[None, :]
    xi = xi.reshape(NUM_WORKERS, N_CHUNKS, CHUNK, NUM_FEATURES)
    xi = xi.transpose(0, 3, 1, 2)
    return _sc_encode(xi, tables.reshape(TAB_ROWS, HIDDEN))


# speedup vs baseline: 1.0059x; 1.0059x over previous
import functools

import jax
import jax.numpy as jnp
from jax import lax
from jax.experimental import pallas as pl
from jax.experimental.pallas import tpu as pltpu
from jax.experimental.pallas import tpu_sc as plsc

BATCH = 16384
NUM_FEATURES = 10
NUM_VALUES = 500
HIDDEN = 128

NUM_CORES = 2
NUM_SUBCORES = 16
NUM_WORKERS = NUM_CORES * NUM_SUBCORES
B_PER_W = BATCH // NUM_WORKERS
CHUNK = 128
N_CHUNKS = B_PER_W // CHUNK
LANES = 16
VECS_PER_ROW = HIDDEN // LANES
GROUPS_PER_CHUNK = CHUNK // LANES

TAB_ROWS = NUM_FEATURES * NUM_VALUES
STAGE_ROWS = TAB_ROWS // NUM_SUBCORES
STAGE_TAIL = TAB_ROWS - (NUM_SUBCORES - 1) * STAGE_ROWS


def _sc_encode(xi, tab):
    mesh = plsc.VectorSubcoreMesh(core_axis_name="c", subcore_axis_name="s")

    @functools.partial(
        pl.kernel,
        mesh=mesh,
        out_type=jax.ShapeDtypeStruct((BATCH, HIDDEN), jnp.float32),
        scratch_types=[
            pltpu.VMEM((NUM_FEATURES, N_CHUNKS, CHUNK), jnp.int32),
            pltpu.VMEM((3, CHUNK, HIDDEN), jnp.float32),
            pltpu.VMEM_SHARED((TAB_ROWS, HIDDEN), jnp.float32),
            pltpu.SemaphoreType.DMA,
            pltpu.SemaphoreType.DMA,
            pltpu.SemaphoreType.DMA((3,)),
            pltpu.SemaphoreType.DMA((3,)),
        ],
    )
    def k(xi_hbm, tab_hbm, out_hbm, idx_all, acc2, shared_tab,
          ssem, hsem, gsem, osem):
        wid = lax.axis_index("s") * NUM_CORES + lax.axis_index("c")
        sid = lax.axis_index("s")
        base = wid * B_PER_W

        @pl.when(sid < NUM_SUBCORES - 1)
        def _():
            pltpu.async_copy(
                tab_hbm.at[pl.ds(sid * STAGE_ROWS, STAGE_ROWS)],
                shared_tab.at[pl.ds(sid * STAGE_ROWS, STAGE_ROWS)],
                ssem,
            )

        @pl.when(sid == NUM_SUBCORES - 1)
        def _():
            pltpu.async_copy(
                tab_hbm.at[pl.ds((NUM_SUBCORES - 1) * STAGE_ROWS, STAGE_TAIL)],
                shared_tab.at[
                    pl.ds((NUM_SUBCORES - 1) * STAGE_ROWS, STAGE_TAIL)],
                ssem,
            )

        pltpu.sync_copy(xi_hbm.at[wid], idx_all)

        zero16 = jnp.zeros((LANES,), jnp.float32)

        def zero_acc(b):
            def zrow(i, _):
                for j in range(VECS_PER_ROW):
                    acc2.at[b][i, pl.ds(j * LANES, LANES)] = zero16
                return 0

            lax.fori_loop(0, CHUNK, zrow, 0)

        def fire_hbm_gathers(cc, b):
            def feat(f, _):
                pltpu.async_copy(
                    tab_hbm.at[idx_all.at[f, cc]], acc2.at[b], hsem,
                    add=True,
                )
                return 0

            lax.fori_loop(0, NUM_FEATURES, feat, 0)

        def fire_gathers(cc, b):
            def feat(f, _):
                pltpu.async_copy(
                    shared_tab.at[idx_all.at[f, cc]], acc2.at[b], gsem.at[b],
                    add=True,
                )
                return 0

            lax.fori_loop(0, NUM_FEATURES, feat, 0)

        def drain_gathers(cc, b):
            def feat_h(f, _):
                pltpu.make_async_copy(
                    tab_hbm.at[idx_all.at[f, cc]], acc2.at[b], hsem
                ).wait()
                return 0

            def feat_s(f, _):
                pltpu.make_async_copy(
                    shared_tab.at[idx_all.at[f, cc]], acc2.at[b], gsem.at[b]
                ).wait()
                return 0

            @pl.when(cc == 0)
            def _():
                lax.fori_loop(0, NUM_FEATURES, feat_h, 0)

            @pl.when(cc > 0)
            def _():
                lax.fori_loop(0, NUM_FEATURES, feat_s, 0)

        zero_acc(0)
        fire_hbm_gathers(0, 0)

        @pl.when(sid < NUM_SUBCORES - 1)
        def _():
            pltpu.make_async_copy(
                tab_hbm.at[pl.ds(sid * STAGE_ROWS, STAGE_ROWS)],
                shared_tab.at[pl.ds(sid * STAGE_ROWS, STAGE_ROWS)],
                ssem,
            ).wait()

        @pl.when(sid == NUM_SUBCORES - 1)
        def _():
            pltpu.make_async_copy(
                tab_hbm.at[pl.ds((NUM_SUBCORES - 1) * STAGE_ROWS, STAGE_TAIL)],
                shared_tab.at[
                    pl.ds((NUM_SUBCORES - 1) * STAGE_ROWS, STAGE_TAIL)],
                ssem,
            ).wait()

        plsc.subcore_barrier()

        zero_acc(1)
        fire_gathers(1, 1)
        zero_acc(2)
        fire_gathers(2, 2)

        def chunk_body(c, _):
            p = c % 3
            drain_gathers(c, p)
            pltpu.async_copy(
                acc2.at[p], out_hbm.at[pl.ds(base + c * CHUNK, CHUNK)],
                osem.at[p],
            )

            @pl.when(c + 3 < N_CHUNKS + 0)
            def _prep_next():
                pltpu.make_async_copy(
                    acc2.at[p], out_hbm.at[pl.ds(base + c * CHUNK, CHUNK)],
                    osem.at[p],
                ).wait()
                zero_acc(p)
                fire_gathers(c + 3, p)

            return 0

        lax.fori_loop(0, N_CHUNKS, chunk_body, 0)

        for c in (N_CHUNKS - 3, N_CHUNKS - 2, N_CHUNKS - 1):
            pltpu.make_async_copy(
                acc2.at[c % 3],
                out_hbm.at[pl.ds(base + c * CHUNK, CHUNK)],
                osem.at[c % 3],
            ).wait()

    return k(xi, tab)


def kernel(x, tables):
    if x.ndim == 1:
        x = x[:, None]
    xi = x.astype(jnp.int32) + NUM_VALUES * jnp.arange(
        NUM_FEATURES, dtype=jnp.int32)[None, :]
    xi = xi.reshape(NUM_WORKERS, N_CHUNKS, CHUNK, NUM_FEATURES)
    xi = xi.transpose(0, 3, 1, 2)
    return _sc_encode(xi, tables.reshape(TAB_ROWS, HIDDEN))
